# baseline (speedup 1.0000x reference)
import functools
import jax
import jax.numpy as jnp
from jax import lax
from jax.experimental import pallas as pl
from jax.experimental.pallas import tpu as pltpu
from jax.experimental.pallas import tpu_sc as plsc

_NSUB = 16
_L = 16
_BIG = 3.0e38


def _splat(v):
    return jnp.full((_L,), v, jnp.float32)


def _perm(x, idx):
    return x.at[idx].get(mode="promise_in_bounds")


def _lane_all(x, op):
    iota = lax.iota(jnp.int32, _L)
    for k in (8, 4, 2, 1):
        x = op(x, _perm(x, jnp.bitwise_xor(iota, k)))
    return x


def _bcast_lane(vec, lane):
    return _perm(vec, jnp.full((_L,), lane, jnp.int32))


def _sc_chamfer(sx, sy, sz, tx, ty, tz, rows_out, cols_out,
                txv, tyv, tzv, sxv, syv, szv, colv, iobuf):
    cid = lax.axis_index("c")
    sid = lax.axis_index("s")
    n = tx.shape[1]
    chunk = n // _NSUB
    base = sid * chunk
    nvec = n // _L
    groups = chunk // 4
    b = cid + 2

    pltpu.sync_copy(tx.at[b], txv)
    pltpu.sync_copy(ty.at[b], tyv)
    pltpu.sync_copy(tz.at[b], tzv)
    pltpu.sync_copy(sx.at[b, pl.ds(base, chunk)], sxv)
    pltpu.sync_copy(sy.at[b, pl.ds(base, chunk)], syv)
    pltpu.sync_copy(sz.at[b, pl.ds(base, chunk)], szv)

    def init_col(j, carry):
        colv[pl.ds(j * _L, _L)] = _splat(_BIG)
        return carry

    lax.fori_loop(0, nvec, init_col, 0)

    def src_group(g, rs):
        i0 = g * 4
        cbase = (i0 // _L) * _L
        off = i0 % _L
        sxc = sxv[pl.ds(cbase, _L)]
        syc = syv[pl.ds(cbase, _L)]
        szc = szv[pl.ds(cbase, _L)]

        s_coords = []
        for c in range(4):
            lane = off + c
            s_coords.append((_bcast_lane(sxc, lane),
                             _bcast_lane(syc, lane),
                             _bcast_lane(szc, lane)))

        def sweep(j, mins):
            o = j * _L
            txr = txv[pl.ds(o, _L)]
            tyr = tyv[pl.ds(o, _L)]
            tzr = tzv[pl.ds(o, _L)]
            ds = []
            new_mins = []
            for c in range(4):
                cx, cy, cz = s_coords[c]
                dx = cx - txr
                dy = cy - tyr
                dz = cz - tzr
                d = dx * dx + dy * dy + dz * dz
                ds.append(d)
                new_mins.append(jnp.minimum(mins[c], d))
            dmin = jnp.minimum(jnp.minimum(ds[0], ds[1]),
                               jnp.minimum(ds[2], ds[3]))
            colv[pl.ds(o, _L)] = jnp.minimum(colv[pl.ds(o, _L)], dmin)
            return tuple(new_mins)

        mins = lax.fori_loop(
            0, nvec, sweep,
            (_splat(_BIG), _splat(_BIG), _splat(_BIG), _splat(_BIG)))
        for c in range(4):
            rs = rs + _lane_all(mins[c], jnp.minimum)
        return rs

    rs = lax.fori_loop(0, groups, src_group, _splat(0.0))

    iobuf[...] = rs
    pltpu.sync_copy(iobuf, rows_out.at[cid, sid])
    pltpu.sync_copy(colv, cols_out.at[cid, sid])


def _tc_pair_kernel(src_ref, tgt_ref, out_src_ref, out_dst_ref):
    b = pl.program_id(0)

    s = src_ref[0]
    t = tgt_ref[0]
    tt = t.T

    sx = s[:, 0:1]
    sy = s[:, 1:2]
    sz = s[:, 2:3]
    tx = tt[0:1, :]
    ty = tt[1:2, :]
    tz = tt[2:3, :]

    dx = sx - tx
    dy = sy - ty
    dz = sz - tz
    dist = dx * dx + dy * dy + dz * dz

    row_min = jnp.min(dist, axis=1, keepdims=True)
    col_min = jnp.min(dist, axis=0, keepdims=True)

    src_sum = jnp.sum(row_min, axis=0, keepdims=True)
    dst_sum = jnp.sum(col_min, axis=1, keepdims=True)

    @pl.when(b == 0)
    def _init():
        out_src_ref[...] = jnp.zeros_like(out_src_ref)
        out_dst_ref[...] = jnp.zeros_like(out_dst_ref)

    out_src_ref[...] += src_sum
    out_dst_ref[...] += dst_sum


def _merge_kernel(tc_src_ref, tc_dst_ref, rows_ref, cols_ref,
                  out_src_ref, out_dst_ref, *, total_src, total_dst):
    rows = rows_ref[...]
    cols = cols_ref[...]

    sc_rows = jnp.sum(rows[:, :, 0:1])
    merged = jnp.min(cols, axis=1)
    sc_cols = jnp.sum(merged)

    out_src_ref[...] = (tc_src_ref[...] + sc_rows) * (1.0 / total_src)
    out_dst_ref[...] = (tc_dst_ref[...] + sc_cols) * (1.0 / total_dst)


def kernel(source, target):
    B, N, _ = source.shape
    M = target.shape[1]

    sx = source[:, :, 0]
    sy = source[:, :, 1]
    sz = source[:, :, 2]
    tx = target[:, :, 0]
    ty = target[:, :, 1]
    tz = target[:, :, 2]

    mesh = plsc.VectorSubcoreMesh(core_axis_name="c", subcore_axis_name="s")
    sc_call = pl.kernel(
        _sc_chamfer,
        mesh=mesh,
        out_type=[
            jax.ShapeDtypeStruct((2, _NSUB, _L), jnp.float32),
            jax.ShapeDtypeStruct((2, _NSUB, M), jnp.float32),
        ],
        scratch_types=[
            pltpu.VMEM((M,), jnp.float32),
            pltpu.VMEM((M,), jnp.float32),
            pltpu.VMEM((M,), jnp.float32),
            pltpu.VMEM((N // _NSUB,), jnp.float32),
            pltpu.VMEM((N // _NSUB,), jnp.float32),
            pltpu.VMEM((N // _NSUB,), jnp.float32),
            pltpu.VMEM((M,), jnp.float32),
            pltpu.VMEM((_L,), jnp.float32),
        ],
    )
    sc_rows, sc_cols = sc_call(sx, sy, sz, tx, ty, tz)

    tc_src, tc_dst = pl.pallas_call(
        _tc_pair_kernel,
        grid=(2,),
        in_specs=[
            pl.BlockSpec((1, N, 3), lambda b: (b, 0, 0)),
            pl.BlockSpec((1, M, 3), lambda b: (b, 0, 0)),
        ],
        out_specs=[
            pl.BlockSpec((1, 1), lambda b: (0, 0)),
            pl.BlockSpec((1, 1), lambda b: (0, 0)),
        ],
        out_shape=[
            jax.ShapeDtypeStruct((1, 1), jnp.float32),
            jax.ShapeDtypeStruct((1, 1), jnp.float32),
        ],
    )(source[:2], target[:2])

    out_src, out_dst = pl.pallas_call(
        functools.partial(_merge_kernel,
                          total_src=float(B * N), total_dst=float(B * M)),
        out_shape=[
            jax.ShapeDtypeStruct((1, 1), jnp.float32),
            jax.ShapeDtypeStruct((1, 1), jnp.float32),
        ],
    )(tc_src, tc_dst, sc_rows, sc_cols)

    return (out_src[0, 0], out_dst[0, 0])


# speedup vs baseline: 1.2989x; 1.2989x over previous
import functools
import jax
import jax.numpy as jnp
from jax import lax
from jax.experimental import pallas as pl
from jax.experimental.pallas import tpu as pltpu
from jax.experimental.pallas import tpu_sc as plsc

_NSUB = 16
_L = 16
_BIG = 3.0e38


def _splat(v):
    return jnp.full((_L,), v, jnp.float32)


def _perm(x, idx):
    return x.at[idx].get(mode="promise_in_bounds")


def _lane_all(x, op):
    iota = lax.iota(jnp.int32, _L)
    for k in (8, 4, 2, 1):
        x = op(x, _perm(x, jnp.bitwise_xor(iota, k)))
    return x


def _bcast_lane(vec, lane):
    return _perm(vec, jnp.full((_L,), lane, jnp.int32))


def _sc_chamfer(sx, sy, sz, tx, ty, tz, rows_out, cols_out,
                txv, tyv, tzv, sxv, syv, szv, colv, iobuf):
    cid = lax.axis_index("c")
    sid = lax.axis_index("s")
    n = tx.shape[1]
    chunk = n // (2 * _NSUB)
    base = (cid * _NSUB + sid) * chunk
    nvec = n // _L
    groups = chunk // 4
    b = 3

    pltpu.sync_copy(tx.at[b], txv)
    pltpu.sync_copy(ty.at[b], tyv)
    pltpu.sync_copy(tz.at[b], tzv)
    pltpu.sync_copy(sx.at[b, pl.ds(base, chunk)], sxv)
    pltpu.sync_copy(sy.at[b, pl.ds(base, chunk)], syv)
    pltpu.sync_copy(sz.at[b, pl.ds(base, chunk)], szv)

    def init_col(j, carry):
        colv[pl.ds(j * _L, _L)] = _splat(_BIG)
        return carry

    lax.fori_loop(0, nvec, init_col, 0)

    def src_group(g, rs):
        i0 = g * 4
        cbase = (i0 // _L) * _L
        off = i0 % _L
        sxc = sxv[pl.ds(cbase, _L)]
        syc = syv[pl.ds(cbase, _L)]
        szc = szv[pl.ds(cbase, _L)]

        s_coords = []
        for c in range(4):
            lane = off + c
            s_coords.append((_bcast_lane(sxc, lane),
                             _bcast_lane(syc, lane),
                             _bcast_lane(szc, lane)))

        def sweep(j, mins):
            o = j * _L
            txr = txv[pl.ds(o, _L)]
            tyr = tyv[pl.ds(o, _L)]
            tzr = tzv[pl.ds(o, _L)]
            ds = []
            new_mins = []
            for c in range(4):
                cx, cy, cz = s_coords[c]
                dx = cx - txr
                dy = cy - tyr
                dz = cz - tzr
                d = dx * dx + dy * dy + dz * dz
                ds.append(d)
                new_mins.append(jnp.minimum(mins[c], d))
            dmin = jnp.minimum(jnp.minimum(ds[0], ds[1]),
                               jnp.minimum(ds[2], ds[3]))
            colv[pl.ds(o, _L)] = jnp.minimum(colv[pl.ds(o, _L)], dmin)
            return tuple(new_mins)

        mins = plsc.parallel_loop(
            0, nvec, unroll=2,
            carry=(_splat(_BIG), _splat(_BIG), _splat(_BIG), _splat(_BIG)),
        )(sweep)
        for c in range(4):
            rs = rs + _lane_all(mins[c], jnp.minimum)
        return rs

    rs = lax.fori_loop(0, groups, src_group, _splat(0.0))

    iobuf[...] = rs
    pltpu.sync_copy(iobuf, rows_out.at[cid, sid])
    pltpu.sync_copy(colv, cols_out.at[cid, sid])


def _tc_pair_kernel(src_ref, tgt_ref, out_src_ref, out_dst_ref):
    b = pl.program_id(0)

    s = src_ref[0]
    t = tgt_ref[0]
    tt = t.T

    sx = s[:, 0:1]
    sy = s[:, 1:2]
    sz = s[:, 2:3]
    tx = tt[0:1, :]
    ty = tt[1:2, :]
    tz = tt[2:3, :]

    dx = sx - tx
    dy = sy - ty
    dz = sz - tz
    dist = dx * dx + dy * dy + dz * dz

    row_min = jnp.min(dist, axis=1, keepdims=True)
    col_min = jnp.min(dist, axis=0, keepdims=True)

    src_sum = jnp.sum(row_min, axis=0, keepdims=True)
    dst_sum = jnp.sum(col_min, axis=1, keepdims=True)

    @pl.when(b == 0)
    def _init():
        out_src_ref[...] = jnp.zeros_like(out_src_ref)
        out_dst_ref[...] = jnp.zeros_like(out_dst_ref)

    out_src_ref[...] += src_sum
    out_dst_ref[...] += dst_sum


def _merge_kernel(tc_src_ref, tc_dst_ref, rows_ref, cols_ref,
                  out_src_ref, out_dst_ref, *, total_src, total_dst):
    rows = rows_ref[...]
    cols = cols_ref[...]

    sc_rows = jnp.sum(rows[:, :, 0:1])
    merged = jnp.min(cols, axis=(0, 1))
    sc_cols = jnp.sum(merged)

    out_src_ref[...] = (tc_src_ref[...] + sc_rows) * (1.0 / total_src)
    out_dst_ref[...] = (tc_dst_ref[...] + sc_cols) * (1.0 / total_dst)


def kernel(source, target):
    B, N, _ = source.shape
    M = target.shape[1]

    sx = source[:, :, 0]
    sy = source[:, :, 1]
    sz = source[:, :, 2]
    tx = target[:, :, 0]
    ty = target[:, :, 1]
    tz = target[:, :, 2]

    mesh = plsc.VectorSubcoreMesh(core_axis_name="c", subcore_axis_name="s")
    sc_call = pl.kernel(
        _sc_chamfer,
        mesh=mesh,
        out_type=[
            jax.ShapeDtypeStruct((2, _NSUB, _L), jnp.float32),
            jax.ShapeDtypeStruct((2, _NSUB, M), jnp.float32),
        ],
        scratch_types=[
            pltpu.VMEM((M,), jnp.float32),
            pltpu.VMEM((M,), jnp.float32),
            pltpu.VMEM((M,), jnp.float32),
            pltpu.VMEM((N // (2 * _NSUB),), jnp.float32),
            pltpu.VMEM((N // (2 * _NSUB),), jnp.float32),
            pltpu.VMEM((N // (2 * _NSUB),), jnp.float32),
            pltpu.VMEM((M,), jnp.float32),
            pltpu.VMEM((_L,), jnp.float32),
        ],
    )
    sc_rows, sc_cols = sc_call(sx, sy, sz, tx, ty, tz)

    tc_src, tc_dst = pl.pallas_call(
        _tc_pair_kernel,
        grid=(3,),
        in_specs=[
            pl.BlockSpec((1, N, 3), lambda b: (b, 0, 0)),
            pl.BlockSpec((1, M, 3), lambda b: (b, 0, 0)),
        ],
        out_specs=[
            pl.BlockSpec((1, 1), lambda b: (0, 0)),
            pl.BlockSpec((1, 1), lambda b: (0, 0)),
        ],
        out_shape=[
            jax.ShapeDtypeStruct((1, 1), jnp.float32),
            jax.ShapeDtypeStruct((1, 1), jnp.float32),
        ],
    )(source[:3], target[:3])

    out_src, out_dst = pl.pallas_call(
        functools.partial(_merge_kernel,
                          total_src=float(B * N), total_dst=float(B * M)),
        out_shape=[
            jax.ShapeDtypeStruct((1, 1), jnp.float32),
            jax.ShapeDtypeStruct((1, 1), jnp.float32),
        ],
    )(tc_src, tc_dst, sc_rows, sc_cols)

    return (out_src[0, 0], out_dst[0, 0])


# speedup vs baseline: 1.2993x; 1.0003x over previous
import functools
import jax
import jax.numpy as jnp
from jax import lax
from jax.experimental import pallas as pl
from jax.experimental.pallas import tpu as pltpu
from jax.experimental.pallas import tpu_sc as plsc

_NSUB = 16
_L = 16
_BIG = 3.0e38


def _splat(v):
    return jnp.full((_L,), v, jnp.float32)


def _perm(x, idx):
    return x.at[idx].get(mode="promise_in_bounds")


def _lane_all(x, op):
    iota = lax.iota(jnp.int32, _L)
    for k in (8, 4, 2, 1):
        x = op(x, _perm(x, jnp.bitwise_xor(iota, k)))
    return x


def _bcast_lane(vec, lane):
    return _perm(vec, jnp.full((_L,), lane, jnp.int32))


def _sc_chamfer(sx, sy, sz, tx, ty, tz, rows_out, cols_out,
                txv, tyv, tzv, sxv, syv, szv, colv, iobuf):
    cid = lax.axis_index("c")
    sid = lax.axis_index("s")
    n = tx.shape[1]
    chunk = n // (2 * _NSUB)
    base = (cid * _NSUB + sid) * chunk
    nvec = n // _L
    groups = chunk // 4
    b = 3

    pltpu.sync_copy(tx.at[b], txv)
    pltpu.sync_copy(ty.at[b], tyv)
    pltpu.sync_copy(tz.at[b], tzv)
    pltpu.sync_copy(sx.at[b, pl.ds(base, chunk)], sxv)
    pltpu.sync_copy(sy.at[b, pl.ds(base, chunk)], syv)
    pltpu.sync_copy(sz.at[b, pl.ds(base, chunk)], szv)

    def init_col(j, carry):
        colv[pl.ds(j * _L, _L)] = _splat(_BIG)
        return carry

    lax.fori_loop(0, nvec, init_col, 0)

    def src_group(g, rs):
        i0 = g * 4
        cbase = (i0 // _L) * _L
        off = i0 % _L
        sxc = sxv[pl.ds(cbase, _L)]
        syc = syv[pl.ds(cbase, _L)]
        szc = szv[pl.ds(cbase, _L)]

        s_coords = []
        for c in range(4):
            lane = off + c
            s_coords.append((_bcast_lane(sxc, lane),
                             _bcast_lane(syc, lane),
                             _bcast_lane(szc, lane)))

        def sweep(j, mins):
            o = j * _L
            txr = txv[pl.ds(o, _L)]
            tyr = tyv[pl.ds(o, _L)]
            tzr = tzv[pl.ds(o, _L)]
            ds = []
            new_mins = []
            for c in range(4):
                cx, cy, cz = s_coords[c]
                dx = cx - txr
                dy = cy - tyr
                dz = cz - tzr
                d = dx * dx + dy * dy + dz * dz
                ds.append(d)
                new_mins.append(jnp.minimum(mins[c], d))
            dmin = jnp.minimum(jnp.minimum(ds[0], ds[1]),
                               jnp.minimum(ds[2], ds[3]))
            colv[pl.ds(o, _L)] = jnp.minimum(colv[pl.ds(o, _L)], dmin)
            return tuple(new_mins)

        mins = plsc.parallel_loop(
            0, nvec, unroll=4,
            carry=(_splat(_BIG), _splat(_BIG), _splat(_BIG), _splat(_BIG)),
        )(sweep)
        for c in range(4):
            rs = rs + _lane_all(mins[c], jnp.minimum)
        return rs

    rs = lax.fori_loop(0, groups, src_group, _splat(0.0))

    iobuf[...] = rs
    pltpu.sync_copy(iobuf, rows_out.at[cid, sid])
    pltpu.sync_copy(colv, cols_out.at[cid, sid])


def _tc_pair_kernel(src_ref, tgt_ref, out_src_ref, out_dst_ref):
    b = pl.program_id(0)

    s = src_ref[0]
    t = tgt_ref[0]
    tt = t.T

    sx = s[:, 0:1]
    sy = s[:, 1:2]
    sz = s[:, 2:3]
    tx = tt[0:1, :]
    ty = tt[1:2, :]
    tz = tt[2:3, :]

    dx = sx - tx
    dy = sy - ty
    dz = sz - tz
    dist = dx * dx + dy * dy + dz * dz

    row_min = jnp.min(dist, axis=1, keepdims=True)
    col_min = jnp.min(dist, axis=0, keepdims=True)

    src_sum = jnp.sum(row_min, axis=0, keepdims=True)
    dst_sum = jnp.sum(col_min, axis=1, keepdims=True)

    @pl.when(b == 0)
    def _init():
        out_src_ref[...] = jnp.zeros_like(out_src_ref)
        out_dst_ref[...] = jnp.zeros_like(out_dst_ref)

    out_src_ref[...] += src_sum
    out_dst_ref[...] += dst_sum


def _merge_kernel(tc_src_ref, tc_dst_ref, rows_ref, cols_ref,
                  out_src_ref, out_dst_ref, *, total_src, total_dst):
    rows = rows_ref[...]
    cols = cols_ref[...]

    sc_rows = jnp.sum(rows[:, :, 0:1])
    merged = jnp.min(cols, axis=(0, 1))
    sc_cols = jnp.sum(merged)

    out_src_ref[...] = (tc_src_ref[...] + sc_rows) * (1.0 / total_src)
    out_dst_ref[...] = (tc_dst_ref[...] + sc_cols) * (1.0 / total_dst)


def kernel(source, target):
    B, N, _ = source.shape
    M = target.shape[1]

    sx = source[:, :, 0]
    sy = source[:, :, 1]
    sz = source[:, :, 2]
    tx = target[:, :, 0]
    ty = target[:, :, 1]
    tz = target[:, :, 2]

    mesh = plsc.VectorSubcoreMesh(core_axis_name="c", subcore_axis_name="s")
    sc_call = pl.kernel(
        _sc_chamfer,
        mesh=mesh,
        out_type=[
            jax.ShapeDtypeStruct((2, _NSUB, _L), jnp.float32),
            jax.ShapeDtypeStruct((2, _NSUB, M), jnp.float32),
        ],
        scratch_types=[
            pltpu.VMEM((M,), jnp.float32),
            pltpu.VMEM((M,), jnp.float32),
            pltpu.VMEM((M,), jnp.float32),
            pltpu.VMEM((N // (2 * _NSUB),), jnp.float32),
            pltpu.VMEM((N // (2 * _NSUB),), jnp.float32),
            pltpu.VMEM((N // (2 * _NSUB),), jnp.float32),
            pltpu.VMEM((M,), jnp.float32),
            pltpu.VMEM((_L,), jnp.float32),
        ],
    )
    sc_rows, sc_cols = sc_call(sx, sy, sz, tx, ty, tz)

    tc_src, tc_dst = pl.pallas_call(
        _tc_pair_kernel,
        grid=(3,),
        in_specs=[
            pl.BlockSpec((1, N, 3), lambda b: (b, 0, 0)),
            pl.BlockSpec((1, M, 3), lambda b: (b, 0, 0)),
        ],
        out_specs=[
            pl.BlockSpec((1, 1), lambda b: (0, 0)),
            pl.BlockSpec((1, 1), lambda b: (0, 0)),
        ],
        out_shape=[
            jax.ShapeDtypeStruct((1, 1), jnp.float32),
            jax.ShapeDtypeStruct((1, 1), jnp.float32),
        ],
    )(source[:3], target[:3])

    out_src, out_dst = pl.pallas_call(
        functools.partial(_merge_kernel,
                          total_src=float(B * N), total_dst=float(B * M)),
        out_shape=[
            jax.ShapeDtypeStruct((1, 1), jnp.float32),
            jax.ShapeDtypeStruct((1, 1), jnp.float32),
        ],
    )(tc_src, tc_dst, sc_rows, sc_cols)

    return (out_src[0, 0], out_dst[0, 0])
